# Initial kernel scaffold; baseline (speedup 1.0000x reference)
import jax
import jax.numpy as jnp
from jax.experimental import pallas as pl


def kernel(x, neighbor_ind, W, ln_gamma, ln_beta):
    raise NotImplementedError("write your pallas kernel here")


# speedup vs baseline: 3.7397x; 3.7397x over previous
import functools
import math

import jax
import jax.numpy as jnp
from jax import lax
from jax.experimental import pallas as pl
from jax.experimental.pallas import tpu as pltpu
from jax.experimental.pallas import tpu_sc as plsc


def _mm_body(x_ref, a_ref, b_ref, y_ref, z_ref):
    xb = x_ref[...]
    dn = (((1,), (0,)), ((), ()))
    y_ref[...] = lax.dot_general(xb, a_ref[...], dn,
                                 precision=lax.Precision.HIGHEST,
                                 preferred_element_type=jnp.float32)
    z_ref[...] = lax.dot_general(xb, b_ref[...], dn,
                                 precision=lax.Precision.HIGHEST,
                                 preferred_element_type=jnp.float32)


def _matmul_yz(x2, a, b):
    nt, d = x2.shape
    bm = 2000
    assert nt % bm == 0
    grid = (nt // bm,)
    return pl.pallas_call(
        _mm_body,
        grid=grid,
        in_specs=[
            pl.BlockSpec((bm, d), lambda i: (i, 0)),
            pl.BlockSpec((d, d), lambda i: (0, 0)),
            pl.BlockSpec((d, d), lambda i: (0, 0)),
        ],
        out_specs=[
            pl.BlockSpec((bm, d), lambda i: (i, 0)),
            pl.BlockSpec((bm, d), lambda i: (i, 0)),
        ],
        out_shape=[
            jax.ShapeDtypeStruct((nt, d), jnp.float32),
            jax.ShapeDtypeStruct((nt, d), jnp.float32),
        ],
    )(x2, a, b)


_CH = 16


def _gather_max(y, idx_flat, npad, k):
    n, d = y.shape
    info = plsc.get_sparse_core_info()
    nw = info.num_cores * info.num_subcores
    rows_per_tile = npad // nw
    nch = rows_per_tile // _CH
    assert rows_per_tile % _CH == 0
    ncg = d // 16

    mesh = plsc.VectorSubcoreMesh(core_axis_name="c", subcore_axis_name="s")

    @functools.partial(
        pl.kernel,
        out_type=jax.ShapeDtypeStruct((npad, d), jnp.float32),
        mesh=mesh,
        scratch_types=[
            pltpu.VMEM((_CH * k,), jnp.int32),
            pltpu.VMEM((_CH * k, d), jnp.float32),
            pltpu.VMEM((_CH, d), jnp.float32),
            pltpu.SemaphoreType.DMA,
        ],
    )
    def gm(y_hbm, idx_hbm, out_hbm, idx_v, rows_v, outc_v, sem):
        cid = lax.axis_index("c")
        sid = lax.axis_index("s")
        wid = sid * info.num_cores + cid
        base = wid * rows_per_tile

        def chunk(g, carry):
            nb = base + g * _CH
            pltpu.sync_copy(idx_hbm.at[pl.ds(nb * k, _CH * k)], idx_v)
            pltpu.async_copy(y_hbm.at[idx_v], rows_v, sem).wait()

            def node(j, carry2):
                for c in range(ncg):
                    m = rows_v[j * k, pl.ds(c * 16, 16)]
                    for r in range(1, k):
                        m = jnp.maximum(m, rows_v[j * k + r, pl.ds(c * 16, 16)])
                    outc_v[j, pl.ds(c * 16, 16)] = m
                return carry2

            lax.fori_loop(0, _CH, node, 0)
            pltpu.sync_copy(outc_v, out_hbm.at[pl.ds(nb, _CH)])
            return carry

        lax.fori_loop(0, nch, chunk, 0)

    return gm(y, idx_flat)


def _post_body(m_ref, z_ref, g_ref, bta_ref, o_ref):
    h = m_ref[...] + z_ref[...]
    mu = jnp.mean(h, axis=-1, keepdims=True)
    var = jnp.mean((h - mu) ** 2, axis=-1, keepdims=True)
    hn = (h - mu) / jnp.sqrt(var + 1e-5) * g_ref[...] + bta_ref[...]
    o_ref[...] = 0.5 * hn * (1.0 + lax.erf(hn * (1.0 / math.sqrt(2.0))))


def _post(m, z, gamma, beta):
    nt, d = m.shape
    bm = 2000
    grid = (nt // bm,)
    return pl.pallas_call(
        _post_body,
        grid=grid,
        in_specs=[
            pl.BlockSpec((bm, d), lambda i: (i, 0)),
            pl.BlockSpec((bm, d), lambda i: (i, 0)),
            pl.BlockSpec((1, d), lambda i: (0, 0)),
            pl.BlockSpec((1, d), lambda i: (0, 0)),
        ],
        out_specs=pl.BlockSpec((bm, d), lambda i: (i, 0)),
        out_shape=jax.ShapeDtypeStruct((nt, d), jnp.float32),
    )(m, z, gamma.reshape(1, d), beta.reshape(1, d))


def kernel(x, neighbor_ind, W, ln_gamma, ln_beta):
    b, n, d = x.shape
    k = neighbor_ind.shape[-1]
    nt = b * n
    x2 = x.reshape(nt, d)

    a = W[:, :d].T
    bb = (W[:, d:] - W[:, :d]).T
    y, z = _matmul_yz(x2, a, bb)

    nw = 32
    npad = ((nt + nw * _CH - 1) // (nw * _CH)) * (nw * _CH)
    idx = neighbor_ind.astype(jnp.int32) + (jnp.arange(b, dtype=jnp.int32) * n)[:, None, None]
    idx = jnp.pad(idx.reshape(nt * k), (0, (npad - nt) * k))

    m = _gather_max(y, idx, npad, k)[:nt]
    out = _post(m, z, ln_gamma, ln_beta)
    return out.reshape(b, n, d)


# speedup vs baseline: 4.3622x; 1.1664x over previous
import functools
import math

import jax
import jax.numpy as jnp
from jax import lax
from jax.experimental import pallas as pl
from jax.experimental.pallas import tpu as pltpu
from jax.experimental.pallas import tpu_sc as plsc


def _mm_body(x_ref, a_ref, b_ref, y_ref, z_ref):
    xb = x_ref[...]
    dn = (((1,), (0,)), ((), ()))
    y_ref[...] = lax.dot_general(xb, a_ref[...], dn,
                                 precision=lax.Precision.HIGHEST,
                                 preferred_element_type=jnp.float32)
    z_ref[...] = lax.dot_general(xb, b_ref[...], dn,
                                 precision=lax.Precision.HIGHEST,
                                 preferred_element_type=jnp.float32)


def _matmul_yz(x2, a, b):
    nt, d = x2.shape
    bm = 2000
    assert nt % bm == 0
    grid = (nt // bm,)
    return pl.pallas_call(
        _mm_body,
        grid=grid,
        in_specs=[
            pl.BlockSpec((bm, d), lambda i: (i, 0)),
            pl.BlockSpec((d, d), lambda i: (0, 0)),
            pl.BlockSpec((d, d), lambda i: (0, 0)),
        ],
        out_specs=[
            pl.BlockSpec((bm, d), lambda i: (i, 0)),
            pl.BlockSpec((bm, d), lambda i: (i, 0)),
        ],
        out_shape=[
            jax.ShapeDtypeStruct((nt, d), jnp.float32),
            jax.ShapeDtypeStruct((nt, d), jnp.float32),
        ],
    )(x2, a, b)


_CH = 16


def _gather_max(y, idx_flat, npad, k):
    n, d = y.shape
    info = plsc.get_sparse_core_info()
    nw = info.num_cores * info.num_subcores
    rows_per_tile = npad // nw
    nch = rows_per_tile // _CH
    assert rows_per_tile % _CH == 0 and nch % 2 == 0
    ncg = d // 16

    mesh = plsc.VectorSubcoreMesh(core_axis_name="c", subcore_axis_name="s")

    @functools.partial(
        pl.kernel,
        out_type=jax.ShapeDtypeStruct((npad, d), jnp.float32),
        mesh=mesh,
        scratch_types=[
            pltpu.VMEM((_CH * k,), jnp.int32),
            pltpu.VMEM((_CH * k,), jnp.int32),
            pltpu.VMEM((_CH * k, d), jnp.float32),
            pltpu.VMEM((_CH * k, d), jnp.float32),
            pltpu.VMEM((_CH, d), jnp.float32),
            pltpu.SemaphoreType.DMA,
            pltpu.SemaphoreType.DMA,
        ],
    )
    def gm(y_hbm, idx_hbm, out_hbm, i0, i1, r0, r1, outc_v, s0, s1):
        cid = lax.axis_index("c")
        sid = lax.axis_index("s")
        wid = sid * info.num_cores + cid
        base = wid * rows_per_tile
        idxs = (i0, i1)
        rows = (r0, r1)
        sems = (s0, s1)

        def start(g, slot):
            nb = base + g * _CH
            pltpu.sync_copy(idx_hbm.at[pl.ds(nb * k, _CH * k)], idxs[slot])
            pltpu.async_copy(y_hbm.at[idxs[slot]], rows[slot], sems[slot])

        def wait(g, slot):
            pltpu.make_async_copy(y_hbm.at[idxs[slot]], rows[slot],
                                  sems[slot]).wait()

        start(0, 0)

        @pl.loop(0, nch, step=2)
        def pair(g0):
            for bs in range(2):
                g = g0 + bs
                nslot = 1 - bs

                @pl.when(g + 1 < nch)
                def _():
                    start(g + 1, nslot)

                wait(g, bs)
                rv = rows[bs]

                def node(j, carry2):
                    for c in range(ncg):
                        m = rv[j * k, pl.ds(c * 16, 16)]
                        for r in range(1, k):
                            m = jnp.maximum(m, rv[j * k + r, pl.ds(c * 16, 16)])
                        outc_v[j, pl.ds(c * 16, 16)] = m
                    return carry2

                lax.fori_loop(0, _CH, node, 0)
                pltpu.sync_copy(outc_v, out_hbm.at[pl.ds(base + g * _CH, _CH)])

    return gm(y, idx_flat)


def _post_body(m_ref, z_ref, g_ref, bta_ref, o_ref):
    h = m_ref[...] + z_ref[...]
    mu = jnp.mean(h, axis=-1, keepdims=True)
    var = jnp.mean((h - mu) ** 2, axis=-1, keepdims=True)
    hn = (h - mu) / jnp.sqrt(var + 1e-5) * g_ref[...] + bta_ref[...]
    o_ref[...] = 0.5 * hn * (1.0 + lax.erf(hn * (1.0 / math.sqrt(2.0))))


def _post(m, z, gamma, beta):
    nt, d = m.shape
    bm = 2000
    grid = (nt // bm,)
    return pl.pallas_call(
        _post_body,
        grid=grid,
        in_specs=[
            pl.BlockSpec((bm, d), lambda i: (i, 0)),
            pl.BlockSpec((bm, d), lambda i: (i, 0)),
            pl.BlockSpec((1, d), lambda i: (0, 0)),
            pl.BlockSpec((1, d), lambda i: (0, 0)),
        ],
        out_specs=pl.BlockSpec((bm, d), lambda i: (i, 0)),
        out_shape=jax.ShapeDtypeStruct((nt, d), jnp.float32),
    )(m, z, gamma.reshape(1, d), beta.reshape(1, d))


def kernel(x, neighbor_ind, W, ln_gamma, ln_beta):
    b, n, d = x.shape
    k = neighbor_ind.shape[-1]
    nt = b * n
    x2 = x.reshape(nt, d)

    a = W[:, :d].T
    bb = (W[:, d:] - W[:, :d]).T
    y, z = _matmul_yz(x2, a, bb)

    nw = 32
    npad = ((nt + nw * _CH - 1) // (nw * _CH)) * (nw * _CH)
    idx = neighbor_ind.astype(jnp.int32) + (jnp.arange(b, dtype=jnp.int32) * n)[:, None, None]
    idx = jnp.pad(idx.reshape(nt * k), (0, (npad - nt) * k))

    m = _gather_max(y, idx, npad, k)[:nt]
    out = _post(m, z, ln_gamma, ln_beta)
    return out.reshape(b, n, d)
